# Initial kernel scaffold; baseline (speedup 1.0000x reference)
import jax
import jax.numpy as jnp
from jax.experimental import pallas as pl


def kernel(features, edge_index, W0, b0, W1, b1, W2, b2):
    raise NotImplementedError("write your pallas kernel here")


# speedup vs baseline: 9.1608x; 9.1608x over previous
import functools

import jax
import jax.numpy as jnp
from jax import lax
from jax.experimental import pallas as pl
from jax.experimental.pallas import tpu as pltpu
from jax.experimental.pallas import tpu_sc as plsc

N = 10000
NP = 10240
E = 320000
NC = 2
NS = 16
EPT = E // (NC * NS)
ROWS_PER_TILE = NP // NS
ZR = 128
C = 80
N_CHUNKS = EPT // C


def _make_sc_seg_sum(F, gather):
    mesh = plsc.VectorSubcoreMesh(core_axis_name="c", subcore_axis_name="s")

    scratch = [
        pltpu.VMEM_SHARED((NP, F), jnp.float32),
        pltpu.VMEM((C,), jnp.int32),
        pltpu.VMEM((C,), jnp.int32),
        pltpu.VMEM((C, F), jnp.float32),
        pltpu.VMEM((ZR, F), jnp.float32),
        pltpu.SemaphoreType.DMA,
    ]

    def body(*refs):
        if gather:
            (y_hbm, src_hbm, dst_hbm, out_hbm,
             acc, sidx, didx, rows, zbuf, sem) = refs
        else:
            (dst_hbm, out_hbm,
             acc, sidx, didx, rows, zbuf, sem) = refs
            y_hbm = src_hbm = None
        cid = lax.axis_index("c")
        sid = lax.axis_index("s")

        zero = jnp.zeros((16,), jnp.float32)
        one = jnp.ones((16,), jnp.float32)

        def fill_z(r, carry):
            for j in range(F // 16):
                zbuf[r, pl.ds(j * 16, 16)] = zero
            return carry

        lax.fori_loop(0, ZR, fill_z, 0)

        if not gather:
            def fill_one(r, carry):
                for j in range(F // 16):
                    rows[r, pl.ds(j * 16, 16)] = one
                return carry

            lax.fori_loop(0, C, fill_one, 0)

        rbase = sid * ROWS_PER_TILE
        for t in range(ROWS_PER_TILE // ZR):
            pltpu.sync_copy(zbuf, acc.at[pl.ds(rbase + t * ZR, ZR)])
        plsc.subcore_barrier()

        ebase = (cid * NS + sid) * EPT

        def chunk(i, carry):
            off = ebase + i * C
            pltpu.sync_copy(dst_hbm.at[pl.ds(off, C)], didx)
            if gather:
                pltpu.sync_copy(src_hbm.at[pl.ds(off, C)], sidx)
                pltpu.async_copy(y_hbm.at[sidx], rows, sem).wait()
            pltpu.sync_copy(rows, acc.at[didx], add=True)
            return carry

        lax.fori_loop(0, N_CHUNKS, chunk, 0)
        plsc.subcore_barrier()

        pltpu.sync_copy(acc.at[pl.ds(rbase, ROWS_PER_TILE)],
                        out_hbm.at[cid, pl.ds(rbase, ROWS_PER_TILE)])

    out_type = jax.ShapeDtypeStruct((NC, NP, F), jnp.float32)
    return pl.kernel(body, mesh=mesh, out_type=out_type,
                     scratch_types=scratch,
                     compiler_params=pltpu.CompilerParams(
                         use_tc_tiling_on_sc=False))


_sc_seg_sum = functools.cache(_make_sc_seg_sum)

_R = 2000


def _dinv(dg0, dg1):
    deg = dg0[:, :1] + dg1[:, :1]
    return jnp.where(deg > 0, lax.rsqrt(deg), 0.0)


def _tc_first_body(x_ref, w_ref, dg0_ref, dg1_ref, y_ref):
    d = _dinv(dg0_ref[...], dg1_ref[...])
    y_ref[...] = d * jnp.dot(x_ref[...], w_ref[...],
                             preferred_element_type=jnp.float32)


def _tc_mid_body(p0_ref, p1_ref, dg0_ref, dg1_ref, b_ref, w_ref, y_ref):
    d = _dinv(dg0_ref[...], dg1_ref[...])
    h = jax.nn.relu(d * (p0_ref[...] + p1_ref[...]) + b_ref[...])
    y_ref[...] = d * jnp.dot(h, w_ref[...],
                             preferred_element_type=jnp.float32)


def _tc_last_body(p0_ref, p1_ref, dg0_ref, dg1_ref, b_ref, y_ref):
    d = _dinv(dg0_ref[...], dg1_ref[...])
    y_ref[...] = jax.nn.relu(d * (p0_ref[...] + p1_ref[...]) + b_ref[...])


def _row_spec(F):
    return pl.BlockSpec((_R, F), lambda i: (i, 0))


def _whole_spec(shape):
    return pl.BlockSpec(shape, lambda i: tuple(0 for _ in shape))


def _tc_first(x, w, dg0, dg1):
    fin, fout = w.shape
    return pl.pallas_call(
        _tc_first_body,
        grid=(N // _R,),
        in_specs=[_row_spec(fin), _whole_spec((fin, fout)),
                  _row_spec(16), _row_spec(16)],
        out_specs=_row_spec(fout),
        out_shape=jax.ShapeDtypeStruct((N, fout), jnp.float32),
    )(x, w, dg0, dg1)


def _tc_mid(p0, p1, dg0, dg1, b, w):
    fin, fout = w.shape
    return pl.pallas_call(
        _tc_mid_body,
        grid=(N // _R,),
        in_specs=[_row_spec(fin), _row_spec(fin),
                  _row_spec(16), _row_spec(16),
                  _whole_spec((1, fin)), _whole_spec((fin, fout))],
        out_specs=_row_spec(fout),
        out_shape=jax.ShapeDtypeStruct((N, fout), jnp.float32),
    )(p0, p1, dg0, dg1, b, w)


def _tc_last(p0, p1, dg0, dg1, b):
    fout = p0.shape[1]
    return pl.pallas_call(
        _tc_last_body,
        grid=(N // _R,),
        in_specs=[_row_spec(fout), _row_spec(fout),
                  _row_spec(16), _row_spec(16),
                  _whole_spec((1, fout))],
        out_specs=_row_spec(fout),
        out_shape=jax.ShapeDtypeStruct((N, fout), jnp.float32),
    )(p0, p1, dg0, dg1, b)


@jax.jit
def kernel(features, edge_index, W0, b0, W1, b1, W2, b2):
    src = edge_index[0].astype(jnp.int32)
    dst = edge_index[1].astype(jnp.int32)

    degp = _sc_seg_sum(16, False)(dst)
    dg0, dg1 = degp[0, :N], degp[1, :N]

    y0 = _tc_first(features, W0, dg0, dg1)
    s0 = _sc_seg_sum(128, True)(y0, src, dst)
    y1 = _tc_mid(s0[0, :N], s0[1, :N], dg0, dg1, b0.reshape(1, -1), W1)
    s1 = _sc_seg_sum(128, True)(y1, src, dst)
    y2 = _tc_mid(s1[0, :N], s1[1, :N], dg0, dg1, b1.reshape(1, -1), W2)
    s2 = _sc_seg_sum(64, True)(y2, src, dst)
    return _tc_last(s2[0, :N], s2[1, :N], dg0, dg1, b2.reshape(1, -1))


# speedup vs baseline: 17.2072x; 1.8784x over previous
import functools

import jax
import jax.numpy as jnp
from jax import lax
from jax.experimental import pallas as pl
from jax.experimental.pallas import tpu as pltpu
from jax.experimental.pallas import tpu_sc as plsc

N = 10000
NP = 10240
E = 320000
NC = 2
NS = 16
EPT = E // (NC * NS)
ROWS_PER_TILE = NP // NS
ZR = 128
C = 80
CH = EPT // C
NB = 2


def _make_sc_seg_sum(F, gather):
    mesh = plsc.VectorSubcoreMesh(core_axis_name="c", subcore_axis_name="s")
    nrows = NB if gather else 1

    scratch = (
        [pltpu.VMEM_SHARED((NP, F), jnp.float32)]
        + [pltpu.VMEM((CH, C), jnp.int32)]
        + ([pltpu.VMEM((CH, C), jnp.int32)] if gather else [])
        + [pltpu.VMEM((C, F), jnp.float32) for _ in range(nrows)]
        + [pltpu.SemaphoreType.DMA for _ in range(NB * (2 if gather else 1))]
    )

    def body(*refs):
        it = iter(refs)
        if gather:
            y_hbm = next(it)
            src2 = next(it)
        dst2 = next(it)
        out_hbm = next(it)
        acc = next(it)
        didx = next(it)
        sidx = next(it) if gather else None
        rows = [next(it) for _ in range(nrows)]
        ssem = [next(it) for _ in range(NB)]
        gsem = [next(it) for _ in range(NB)] if gather else None

        cid = lax.axis_index("c")
        sid = lax.axis_index("s")

        zero = jnp.zeros((16,), jnp.float32)
        one = jnp.ones((16,), jnp.float32)

        def fill(buf, val):
            def fill_row(r, carry):
                for j in range(F // 16):
                    buf[r, pl.ds(j * 16, 16)] = val
                return carry

            lax.fori_loop(0, C, fill_row, 0)

        fill(rows[0], zero)
        rbase = sid * ROWS_PER_TILE
        for t in range(ROWS_PER_TILE // C):
            pltpu.sync_copy(rows[0], acc.at[pl.ds(rbase + t * C, C)])
        if not gather:
            fill(rows[0], one)
        plsc.subcore_barrier()

        cbase = (cid * NS + sid) * CH
        pltpu.sync_copy(dst2.at[pl.ds(cbase, CH)], didx)
        if gather:
            pltpu.sync_copy(src2.at[pl.ds(cbase, CH)], sidx)

        def sstart(i, r, rr):
            pltpu.async_copy(rows[rr], acc.at[didx.at[i]], ssem[r], add=True)

        def swait(r, rr):
            pltpu.make_async_copy(rows[rr], acc.at[didx.at[0]],
                                  ssem[r]).wait()

        if gather:
            def gstart(i, r):
                pltpu.async_copy(y_hbm.at[sidx.at[i]], rows[r], gsem[r])

            def gwait(r):
                pltpu.make_async_copy(y_hbm.at[sidx.at[0]], rows[r],
                                      gsem[r]).wait()

            def step(i, r):
                gwait(r)
                sstart(i, r, r)
                swait(1 - r, 1 - r)
                gstart(i + 1, 1 - r)

            gstart(0, 0)
            gwait(0)
            sstart(0, 0, 0)
            gstart(1, 1)

            def kblock(k, carry):
                i = 1 + 2 * k
                step(i, 1)
                step(i + 1, 0)
                return carry

            lax.fori_loop(0, (CH - 3) // 2, kblock, 0)
            step(CH - 2, 1)
            gwait(0)
            sstart(CH - 1, 0, 0)
            swait(1, 1)
            swait(0, 0)
        else:
            sstart(0, 0, 0)
            sstart(1, 1, 0)

            def kblock(k, carry):
                i = 2 * k
                swait(0, 0)
                sstart(i, 0, 0)
                swait(1, 0)
                sstart(i + 1, 1, 0)
                return carry

            lax.fori_loop(1, (CH - 1) // 2, kblock, 0)
            swait(0, 0)
            sstart(CH - 1, 0, 0)
            swait(1, 0)
            swait(0, 0)

        plsc.subcore_barrier()

        pltpu.sync_copy(acc.at[pl.ds(rbase, ROWS_PER_TILE)],
                        out_hbm.at[cid, pl.ds(rbase, ROWS_PER_TILE)])

    out_type = jax.ShapeDtypeStruct((NC, NP, F), jnp.float32)
    return pl.kernel(body, mesh=mesh, out_type=out_type,
                     scratch_types=scratch,
                     compiler_params=pltpu.CompilerParams(
                         use_tc_tiling_on_sc=False))


_sc_seg_sum = functools.cache(_make_sc_seg_sum)

_R = 2000


def _dinv(dg0, dg1):
    deg = dg0[:, :1] + dg1[:, :1]
    return jnp.where(deg > 0, lax.rsqrt(deg), 0.0)


def _tc_first_body(x_ref, w_ref, dg0_ref, dg1_ref, y_ref):
    d = _dinv(dg0_ref[...], dg1_ref[...])
    y_ref[...] = d * jnp.dot(x_ref[...], w_ref[...],
                             preferred_element_type=jnp.float32)


def _tc_mid_body(p0_ref, p1_ref, dg0_ref, dg1_ref, b_ref, w_ref, y_ref):
    d = _dinv(dg0_ref[...], dg1_ref[...])
    h = jax.nn.relu(d * (p0_ref[...] + p1_ref[...]) + b_ref[...])
    y_ref[...] = d * jnp.dot(h, w_ref[...],
                             preferred_element_type=jnp.float32)


def _tc_last_body(p0_ref, p1_ref, dg0_ref, dg1_ref, b_ref, y_ref):
    d = _dinv(dg0_ref[...], dg1_ref[...])
    y_ref[...] = jax.nn.relu(d * (p0_ref[...] + p1_ref[...]) + b_ref[...])


def _row_spec(F):
    return pl.BlockSpec((_R, F), lambda i: (i, 0))


def _whole_spec(shape):
    return pl.BlockSpec(shape, lambda i: tuple(0 for _ in shape))


def _tc_first(x, w, dg0, dg1):
    fin, fout = w.shape
    return pl.pallas_call(
        _tc_first_body,
        grid=(N // _R,),
        in_specs=[_row_spec(fin), _whole_spec((fin, fout)),
                  _row_spec(16), _row_spec(16)],
        out_specs=_row_spec(fout),
        out_shape=jax.ShapeDtypeStruct((N, fout), jnp.float32),
    )(x, w, dg0, dg1)


def _tc_mid(p0, p1, dg0, dg1, b, w):
    fin, fout = w.shape
    return pl.pallas_call(
        _tc_mid_body,
        grid=(N // _R,),
        in_specs=[_row_spec(fin), _row_spec(fin),
                  _row_spec(16), _row_spec(16),
                  _whole_spec((1, fin)), _whole_spec((fin, fout))],
        out_specs=_row_spec(fout),
        out_shape=jax.ShapeDtypeStruct((N, fout), jnp.float32),
    )(p0, p1, dg0, dg1, b, w)


def _tc_last(p0, p1, dg0, dg1, b):
    fout = p0.shape[1]
    return pl.pallas_call(
        _tc_last_body,
        grid=(N // _R,),
        in_specs=[_row_spec(fout), _row_spec(fout),
                  _row_spec(16), _row_spec(16),
                  _whole_spec((1, fout))],
        out_specs=_row_spec(fout),
        out_shape=jax.ShapeDtypeStruct((N, fout), jnp.float32),
    )(p0, p1, dg0, dg1, b)


@jax.jit
def kernel(features, edge_index, W0, b0, W1, b1, W2, b2):
    src2 = edge_index[0].astype(jnp.int32).reshape(E // C, C)
    dst2 = edge_index[1].astype(jnp.int32).reshape(E // C, C)

    degp = _sc_seg_sum(16, False)(dst2)
    dg0, dg1 = degp[0, :N], degp[1, :N]

    y0 = _tc_first(features, W0, dg0, dg1)
    s0 = _sc_seg_sum(128, True)(y0, src2, dst2)
    y1 = _tc_mid(s0[0, :N], s0[1, :N], dg0, dg1, b0.reshape(1, -1), W1)
    s1 = _sc_seg_sum(128, True)(y1, src2, dst2)
    y2 = _tc_mid(s1[0, :N], s1[1, :N], dg0, dg1, b1.reshape(1, -1), W2)
    s2 = _sc_seg_sum(64, True)(y2, src2, dst2)
    return _tc_last(s2[0, :N], s2[1, :N], dg0, dg1, b2.reshape(1, -1))


# speedup vs baseline: 18.5719x; 1.0793x over previous
import functools

import jax
import jax.numpy as jnp
from jax import lax
from jax.experimental import pallas as pl
from jax.experimental.pallas import tpu as pltpu
from jax.experimental.pallas import tpu_sc as plsc

N = 10000
NP = 10240
E = 320000
NC = 2
NS = 16
EPT = E // (NC * NS)
ROWS_PER_TILE = NP // NS
ZR = 128
C = 100
CH = EPT // C
NB = 2


def _make_sc_seg_sum(F, gather):
    mesh = plsc.VectorSubcoreMesh(core_axis_name="c", subcore_axis_name="s")
    nrows = NB if gather else 1

    scratch = (
        [pltpu.VMEM_SHARED((NP, F), jnp.float32)]
        + [pltpu.VMEM((CH, C), jnp.int32)]
        + ([pltpu.VMEM((CH, C), jnp.int32)] if gather else [])
        + [pltpu.VMEM((C, F), jnp.float32) for _ in range(nrows)]
        + [pltpu.SemaphoreType.DMA for _ in range(NB * (2 if gather else 1))]
    )

    def body(*refs):
        it = iter(refs)
        if gather:
            y_hbm = next(it)
            src2 = next(it)
        dst2 = next(it)
        out_hbm = next(it)
        acc = next(it)
        didx = next(it)
        sidx = next(it) if gather else None
        rows = [next(it) for _ in range(nrows)]
        ssem = [next(it) for _ in range(NB)]
        gsem = [next(it) for _ in range(NB)] if gather else None

        cid = lax.axis_index("c")
        sid = lax.axis_index("s")

        zero = jnp.zeros((16,), jnp.float32)
        one = jnp.ones((16,), jnp.float32)

        def fill(buf, val):
            def fill_row(r, carry):
                for j in range(F // 16):
                    buf[r, pl.ds(j * 16, 16)] = val
                return carry

            lax.fori_loop(0, C, fill_row, 0)

        fill(rows[0], zero)
        rbase = sid * ROWS_PER_TILE
        for t in range(ROWS_PER_TILE // C):
            pltpu.sync_copy(rows[0], acc.at[pl.ds(rbase + t * C, C)])
        remz = ROWS_PER_TILE % C
        if remz:
            pltpu.sync_copy(
                rows[0].at[pl.ds(0, remz)],
                acc.at[pl.ds(rbase + (ROWS_PER_TILE // C) * C, remz)])
        if not gather:
            fill(rows[0], one)
        plsc.subcore_barrier()

        cbase = (cid * NS + sid) * CH
        pltpu.sync_copy(dst2.at[pl.ds(cbase, CH)], didx)
        if gather:
            pltpu.sync_copy(src2.at[pl.ds(cbase, CH)], sidx)

        def sstart(i, r, rr):
            pltpu.async_copy(rows[rr], acc.at[didx.at[i]], ssem[r], add=True)

        def swait(r, rr):
            pltpu.make_async_copy(rows[rr], acc.at[didx.at[0]],
                                  ssem[r]).wait()

        if gather:
            def gstart(i, r):
                pltpu.async_copy(y_hbm.at[sidx.at[i]], rows[r], gsem[r])

            def gwait(r):
                pltpu.make_async_copy(y_hbm.at[sidx.at[0]], rows[r],
                                      gsem[r]).wait()

            def step(i, r):
                gwait(r)
                sstart(i, r, r)
                swait(1 - r, 1 - r)
                gstart(i + 1, 1 - r)

            gstart(0, 0)
            gwait(0)
            sstart(0, 0, 0)
            gstart(1, 1)
            npairs = (CH - 2) // 2

            def kblock(k, carry):
                i = 1 + 2 * k
                step(i, 1)
                step(i + 1, 0)
                return carry

            lax.fori_loop(0, npairs, kblock, 0)
            for i in range(1 + 2 * npairs, CH - 1):
                step(i, i & 1)
            p = (CH - 1) & 1
            gwait(p)
            sstart(CH - 1, p, p)
            swait(1 - p, 1 - p)
            swait(p, p)
        else:
            sstart(0, 0, 0)
            sstart(1, 1, 0)
            npairs = (CH - 2) // 2

            def kblock(k, carry):
                i = 2 + 2 * k
                swait(0, 0)
                sstart(i, 0, 0)
                swait(1, 0)
                sstart(i + 1, 1, 0)
                return carry

            lax.fori_loop(0, npairs, kblock, 0)
            for i in range(2 + 2 * npairs, CH):
                p = i & 1
                swait(p, 0)
                sstart(i, p, 0)
            swait(1, 0)
            swait(0, 0)

        plsc.subcore_barrier()

        pltpu.sync_copy(acc.at[pl.ds(rbase, ROWS_PER_TILE)],
                        out_hbm.at[cid, pl.ds(rbase, ROWS_PER_TILE)])

    out_type = jax.ShapeDtypeStruct((NC, NP, F), jnp.float32)
    return pl.kernel(body, mesh=mesh, out_type=out_type,
                     scratch_types=scratch,
                     compiler_params=pltpu.CompilerParams(
                         use_tc_tiling_on_sc=False))


_sc_seg_sum = functools.cache(_make_sc_seg_sum)

_R = 2000


def _dinv(dg0, dg1):
    deg = dg0[:, :1] + dg1[:, :1]
    return jnp.where(deg > 0, lax.rsqrt(deg), 0.0)


def _tc_first_body(x_ref, w_ref, dg0_ref, dg1_ref, y_ref):
    d = _dinv(dg0_ref[...], dg1_ref[...])
    y_ref[...] = d * jnp.dot(x_ref[...], w_ref[...],
                             preferred_element_type=jnp.float32)


def _tc_mid_body(p0_ref, p1_ref, dg0_ref, dg1_ref, b_ref, w_ref, y_ref):
    d = _dinv(dg0_ref[...], dg1_ref[...])
    h = jax.nn.relu(d * (p0_ref[...] + p1_ref[...]) + b_ref[...])
    y_ref[...] = d * jnp.dot(h, w_ref[...],
                             preferred_element_type=jnp.float32)


def _tc_last_body(p0_ref, p1_ref, dg0_ref, dg1_ref, b_ref, y_ref):
    d = _dinv(dg0_ref[...], dg1_ref[...])
    y_ref[...] = jax.nn.relu(d * (p0_ref[...] + p1_ref[...]) + b_ref[...])


def _row_spec(F):
    return pl.BlockSpec((_R, F), lambda i: (i, 0))


def _whole_spec(shape):
    return pl.BlockSpec(shape, lambda i: tuple(0 for _ in shape))


def _tc_first(x, w, dg0, dg1):
    fin, fout = w.shape
    return pl.pallas_call(
        _tc_first_body,
        grid=(N // _R,),
        in_specs=[_row_spec(fin), _whole_spec((fin, fout)),
                  _row_spec(16), _row_spec(16)],
        out_specs=_row_spec(fout),
        out_shape=jax.ShapeDtypeStruct((N, fout), jnp.float32),
    )(x, w, dg0, dg1)


def _tc_mid(p0, p1, dg0, dg1, b, w):
    fin, fout = w.shape
    return pl.pallas_call(
        _tc_mid_body,
        grid=(N // _R,),
        in_specs=[_row_spec(fin), _row_spec(fin),
                  _row_spec(16), _row_spec(16),
                  _whole_spec((1, fin)), _whole_spec((fin, fout))],
        out_specs=_row_spec(fout),
        out_shape=jax.ShapeDtypeStruct((N, fout), jnp.float32),
    )(p0, p1, dg0, dg1, b, w)


def _tc_last(p0, p1, dg0, dg1, b):
    fout = p0.shape[1]
    return pl.pallas_call(
        _tc_last_body,
        grid=(N // _R,),
        in_specs=[_row_spec(fout), _row_spec(fout),
                  _row_spec(16), _row_spec(16),
                  _whole_spec((1, fout))],
        out_specs=_row_spec(fout),
        out_shape=jax.ShapeDtypeStruct((N, fout), jnp.float32),
    )(p0, p1, dg0, dg1, b)


@jax.jit
def kernel(features, edge_index, W0, b0, W1, b1, W2, b2):
    src2 = edge_index[0].astype(jnp.int32).reshape(E // C, C)
    dst2 = edge_index[1].astype(jnp.int32).reshape(E // C, C)

    degp = _sc_seg_sum(16, False)(dst2)
    dg0, dg1 = degp[0, :N], degp[1, :N]

    y0 = _tc_first(features, W0, dg0, dg1)
    s0 = _sc_seg_sum(128, True)(y0, src2, dst2)
    y1 = _tc_mid(s0[0, :N], s0[1, :N], dg0, dg1, b0.reshape(1, -1), W1)
    s1 = _sc_seg_sum(128, True)(y1, src2, dst2)
    y2 = _tc_mid(s1[0, :N], s1[1, :N], dg0, dg1, b1.reshape(1, -1), W2)
    s2 = _sc_seg_sum(64, True)(y2, src2, dst2)
    return _tc_last(s2[0, :N], s2[1, :N], dg0, dg1, b2.reshape(1, -1))


# speedup vs baseline: 24.0339x; 1.2941x over previous
import functools

import jax
import jax.numpy as jnp
from jax import lax
from jax.experimental import pallas as pl
from jax.experimental.pallas import tpu as pltpu
from jax.experimental.pallas import tpu_sc as plsc

N = 10000
NP = 10000
E = 320000
NC = 2
NS = 16
EPT = E // (NC * NS)
ROWS_PER_TILE = NP // NS
C = 80
CH = EPT // C
NB = 3


def _make_sc_seg_sum(F, gather):
    mesh = plsc.VectorSubcoreMesh(core_axis_name="c", subcore_axis_name="s")
    nrows = NB if gather else 1

    scratch = (
        [pltpu.VMEM_SHARED((NP, F), jnp.float32)]
        + [pltpu.VMEM((CH, C), jnp.int32)]
        + ([pltpu.VMEM((CH, C), jnp.int32)] if gather else [])
        + [pltpu.VMEM((C, F), jnp.float32) for _ in range(nrows)]
        + [pltpu.SemaphoreType.DMA for _ in range(NB * (2 if gather else 1))]
    )

    def body(*refs):
        it = iter(refs)
        if gather:
            y_hbm = next(it)
            src2 = next(it)
        dst2 = next(it)
        out_hbm = next(it)
        acc = next(it)
        didx = next(it)
        sidx = next(it) if gather else None
        rows = [next(it) for _ in range(nrows)]
        ssem = [next(it) for _ in range(NB)]
        gsem = [next(it) for _ in range(NB)] if gather else None

        cid = lax.axis_index("c")
        sid = lax.axis_index("s")

        zero = jnp.zeros((16,), jnp.float32)
        one = jnp.ones((16,), jnp.float32)

        def fill(buf, val):
            def fill_row(r, carry):
                for j in range(F // 16):
                    buf[r, pl.ds(j * 16, 16)] = val
                return carry

            lax.fori_loop(0, C, fill_row, 0)

        fill(rows[0], zero)
        rbase = sid * ROWS_PER_TILE
        for t in range(ROWS_PER_TILE // C):
            pltpu.sync_copy(rows[0], acc.at[pl.ds(rbase + t * C, C)])
        remz = ROWS_PER_TILE % C
        if remz:
            pltpu.sync_copy(
                rows[0].at[pl.ds(0, remz)],
                acc.at[pl.ds(rbase + (ROWS_PER_TILE // C) * C, remz)])
        if not gather:
            fill(rows[0], one)
        plsc.subcore_barrier()

        cbase = (cid * NS + sid) * CH
        pltpu.sync_copy(dst2.at[pl.ds(cbase, CH)], didx)
        if gather:
            pltpu.sync_copy(src2.at[pl.ds(cbase, CH)], sidx)

        def sstart(i, r, rr):
            pltpu.async_copy(rows[rr], acc.at[didx.at[i]], ssem[r], add=True)

        def swait(r, rr):
            pltpu.make_async_copy(rows[rr], acc.at[didx.at[0]],
                                  ssem[r]).wait()

        if gather:
            def gstart(i, r):
                pltpu.async_copy(y_hbm.at[sidx.at[i]], rows[r], gsem[r])

            def gwait(r):
                pltpu.make_async_copy(y_hbm.at[sidx.at[0]], rows[r],
                                      gsem[r]).wait()

            def step(i, r):
                gwait(r)
                sstart(i, r, r)
                rp = (r + 2) % NB
                swait(rp, rp)
                gstart(i + 2, rp)

            gstart(0, 0)
            gstart(1, 1)
            gwait(0)
            sstart(0, 0, 0)
            gstart(2, 2)
            nblocks = (CH - 3) // 3

            def kblock(k, carry):
                b = 1 + 3 * k
                step(b, 1)
                step(b + 1, 2)
                step(b + 2, 0)
                return carry

            lax.fori_loop(0, nblocks, kblock, 0)
            for i in range(1 + 3 * nblocks, CH - 2):
                step(i, i % NB)
            for i in range(CH - 2, CH):
                gwait(i % NB)
                sstart(i, i % NB, i % NB)
            for i in range(CH - 3, CH):
                swait(i % NB, i % NB)
        else:
            sstart(0, 0, 0)
            sstart(1, 1, 0)
            npairs = (CH - 2) // 2

            def kblock(k, carry):
                i = 2 + 2 * k
                swait(0, 0)
                sstart(i, 0, 0)
                swait(1, 0)
                sstart(i + 1, 1, 0)
                return carry

            lax.fori_loop(0, npairs, kblock, 0)
            for i in range(2 + 2 * npairs, CH):
                p = i & 1
                swait(p, 0)
                sstart(i, p, 0)
            swait(1, 0)
            swait(0, 0)

        plsc.subcore_barrier()

        pltpu.sync_copy(acc.at[pl.ds(rbase, ROWS_PER_TILE)],
                        out_hbm.at[cid, pl.ds(rbase, ROWS_PER_TILE)])

    out_type = jax.ShapeDtypeStruct((NC, NP, F), jnp.float32)
    return pl.kernel(body, mesh=mesh, out_type=out_type,
                     scratch_types=scratch,
                     compiler_params=pltpu.CompilerParams(
                         use_tc_tiling_on_sc=False))


_sc_seg_sum = functools.cache(_make_sc_seg_sum)

_R = 2000


def _dinv(dg0, dg1):
    deg = dg0[:, :1] + dg1[:, :1]
    return jnp.where(deg > 0, lax.rsqrt(deg), 0.0)


def _tc_first_body(x_ref, w_ref, dg0_ref, dg1_ref, y_ref):
    d = _dinv(dg0_ref[...], dg1_ref[...])
    y_ref[...] = d * jnp.dot(x_ref[...], w_ref[...],
                             preferred_element_type=jnp.float32)


def _tc_mid_body(p0_ref, p1_ref, dg0_ref, dg1_ref, b_ref, w_ref, y_ref):
    d = _dinv(dg0_ref[...], dg1_ref[...])
    h = jax.nn.relu(d * (p0_ref[...] + p1_ref[...]) + b_ref[...])
    y_ref[...] = d * jnp.dot(h, w_ref[...],
                             preferred_element_type=jnp.float32)


def _tc_last_body(p0_ref, p1_ref, dg0_ref, dg1_ref, b_ref, y_ref):
    d = _dinv(dg0_ref[...], dg1_ref[...])
    y_ref[...] = jax.nn.relu(d * (p0_ref[...] + p1_ref[...]) + b_ref[...])


def _row_spec(F):
    return pl.BlockSpec((_R, F), lambda i: (i, 0))


def _whole_spec(shape):
    return pl.BlockSpec(shape, lambda i: tuple(0 for _ in shape))


def _tc_first(x, w, dg0, dg1):
    fin, fout = w.shape
    return pl.pallas_call(
        _tc_first_body,
        grid=(N // _R,),
        in_specs=[_row_spec(fin), _whole_spec((fin, fout)),
                  _row_spec(16), _row_spec(16)],
        out_specs=_row_spec(fout),
        out_shape=jax.ShapeDtypeStruct((N, fout), jnp.float32),
    )(x, w, dg0, dg1)


def _tc_mid(p0, p1, dg0, dg1, b, w):
    fin, fout = w.shape
    return pl.pallas_call(
        _tc_mid_body,
        grid=(N // _R,),
        in_specs=[_row_spec(fin), _row_spec(fin),
                  _row_spec(16), _row_spec(16),
                  _whole_spec((1, fin)), _whole_spec((fin, fout))],
        out_specs=_row_spec(fout),
        out_shape=jax.ShapeDtypeStruct((N, fout), jnp.float32),
    )(p0, p1, dg0, dg1, b, w)


def _tc_last(p0, p1, dg0, dg1, b):
    fout = p0.shape[1]
    return pl.pallas_call(
        _tc_last_body,
        grid=(N // _R,),
        in_specs=[_row_spec(fout), _row_spec(fout),
                  _row_spec(16), _row_spec(16),
                  _whole_spec((1, fout))],
        out_specs=_row_spec(fout),
        out_shape=jax.ShapeDtypeStruct((N, fout), jnp.float32),
    )(p0, p1, dg0, dg1, b)


@jax.jit
def kernel(features, edge_index, W0, b0, W1, b1, W2, b2):
    src2 = edge_index[0].astype(jnp.int32).reshape(E // C, C)
    dst2 = edge_index[1].astype(jnp.int32).reshape(E // C, C)

    degp = _sc_seg_sum(16, False)(dst2)
    dg0, dg1 = degp[0, :N], degp[1, :N]

    y0 = _tc_first(features, W0, dg0, dg1)
    s0 = _sc_seg_sum(128, True)(y0, src2, dst2)
    y1 = _tc_mid(s0[0, :N], s0[1, :N], dg0, dg1, b0.reshape(1, -1), W1)
    s1 = _sc_seg_sum(128, True)(y1, src2, dst2)
    y2 = _tc_mid(s1[0, :N], s1[1, :N], dg0, dg1, b1.reshape(1, -1), W2)
    s2 = _sc_seg_sum(64, True)(y2, src2, dst2)
    return _tc_last(s2[0, :N], s2[1, :N], dg0, dg1, b2.reshape(1, -1))


# speedup vs baseline: 26.7784x; 1.1142x over previous
import functools

import jax
import jax.numpy as jnp
from jax import lax
from jax.experimental import pallas as pl
from jax.experimental.pallas import tpu as pltpu
from jax.experimental.pallas import tpu_sc as plsc

N = 10000
NP = 10000
E = 320000
NC = 2
NS = 16
EPT = E // (NC * NS)
ROWS_PER_TILE = NP // NS
C = 80
CH = EPT // C
NB = 3


def _make_sc_seg_sum(F, gather):
    mesh = plsc.VectorSubcoreMesh(core_axis_name="c", subcore_axis_name="s")
    nrows = NB if gather else 1

    scratch = (
        [pltpu.VMEM_SHARED((NP, F), jnp.float32)]
        + [pltpu.VMEM((CH, C), jnp.int32)]
        + ([pltpu.VMEM((CH, C), jnp.int32)] if gather else [])
        + [pltpu.VMEM((C, F), jnp.float32) for _ in range(nrows)]
        + [pltpu.SemaphoreType.DMA for _ in range(NB * (2 if gather else 1))]
    )

    def body(*refs):
        it = iter(refs)
        if gather:
            y_hbm = next(it)
        edge2 = next(it)
        out_hbm = next(it)
        acc = next(it)
        didx = next(it)
        sidx = next(it) if gather else None
        rows = [next(it) for _ in range(nrows)]
        ssem = [next(it) for _ in range(NB)]
        gsem = [next(it) for _ in range(NB)] if gather else None

        cid = lax.axis_index("c")
        sid = lax.axis_index("s")

        zero = jnp.zeros((16,), jnp.float32)
        one = jnp.ones((16,), jnp.float32)

        def fill(buf, val):
            def fill_row(r, carry):
                for j in range(F // 16):
                    buf[r, pl.ds(j * 16, 16)] = val
                return carry

            lax.fori_loop(0, C, fill_row, 0)

        fill(rows[0], zero)
        rbase = sid * ROWS_PER_TILE
        for t in range(ROWS_PER_TILE // C):
            pltpu.sync_copy(rows[0], acc.at[pl.ds(rbase + t * C, C)])
        remz = ROWS_PER_TILE % C
        if remz:
            pltpu.sync_copy(
                rows[0].at[pl.ds(0, remz)],
                acc.at[pl.ds(rbase + (ROWS_PER_TILE // C) * C, remz)])
        if not gather:
            fill(rows[0], one)
        plsc.subcore_barrier()

        cbase = (cid * NS + sid) * CH
        pltpu.sync_copy(edge2.at[1, pl.ds(cbase, CH)], didx)
        if gather:
            pltpu.sync_copy(edge2.at[0, pl.ds(cbase, CH)], sidx)

        def sstart(i, r, rr):
            pltpu.async_copy(rows[rr], acc.at[didx.at[i]], ssem[r], add=True)

        def swait(r, rr):
            pltpu.make_async_copy(rows[rr], acc.at[didx.at[0]],
                                  ssem[r]).wait()

        if gather:
            def gstart(i, r):
                pltpu.async_copy(y_hbm.at[sidx.at[i]], rows[r], gsem[r])

            def gwait(r):
                pltpu.make_async_copy(y_hbm.at[sidx.at[0]], rows[r],
                                      gsem[r]).wait()

            def step(i, r):
                gwait(r)
                sstart(i, r, r)
                rp = (r + 2) % NB
                swait(rp, rp)
                gstart(i + 2, rp)

            gstart(0, 0)
            gstart(1, 1)
            gwait(0)
            sstart(0, 0, 0)
            gstart(2, 2)
            nblocks = (CH - 3) // 3

            def kblock(k, carry):
                b = 1 + 3 * k
                step(b, 1)
                step(b + 1, 2)
                step(b + 2, 0)
                return carry

            lax.fori_loop(0, nblocks, kblock, 0)
            for i in range(1 + 3 * nblocks, CH - 2):
                step(i, i % NB)
            for i in range(CH - 2, CH):
                gwait(i % NB)
                sstart(i, i % NB, i % NB)
            for i in range(CH - 3, CH):
                swait(i % NB, i % NB)
        else:
            sstart(0, 0, 0)
            sstart(1, 1, 0)
            npairs = (CH - 2) // 2

            def kblock(k, carry):
                i = 2 + 2 * k
                swait(0, 0)
                sstart(i, 0, 0)
                swait(1, 0)
                sstart(i + 1, 1, 0)
                return carry

            lax.fori_loop(0, npairs, kblock, 0)
            for i in range(2 + 2 * npairs, CH):
                p = i & 1
                swait(p, 0)
                sstart(i, p, 0)
            swait(1, 0)
            swait(0, 0)

        plsc.subcore_barrier()

        pltpu.sync_copy(acc.at[pl.ds(rbase, ROWS_PER_TILE)],
                        out_hbm.at[cid, pl.ds(rbase, ROWS_PER_TILE)])

    out_type = jax.ShapeDtypeStruct((NC, NP, F), jnp.float32)
    return pl.kernel(body, mesh=mesh, out_type=out_type,
                     scratch_types=scratch,
                     compiler_params=pltpu.CompilerParams(
                         use_tc_tiling_on_sc=False))


_sc_seg_sum = functools.cache(_make_sc_seg_sum)

_R = 2000


def _dinv(dgp):
    deg = dgp[0, :, :1] + dgp[1, :, :1]
    return jnp.where(deg > 0, lax.rsqrt(deg), 0.0)


def _tc_first_body(x_ref, w_ref, dgp_ref, y_ref):
    d = _dinv(dgp_ref[...])
    y_ref[...] = d * jnp.dot(x_ref[...], w_ref[...],
                             preferred_element_type=jnp.float32)


def _tc_mid_body(s_ref, dgp_ref, b_ref, w_ref, y_ref):
    d = _dinv(dgp_ref[...])
    h = jax.nn.relu(d * (s_ref[0] + s_ref[1]) + b_ref[...])
    y_ref[...] = d * jnp.dot(h, w_ref[...],
                             preferred_element_type=jnp.float32)


def _tc_last_body(s_ref, dgp_ref, b_ref, y_ref):
    d = _dinv(dgp_ref[...])
    y_ref[...] = jax.nn.relu(d * (s_ref[0] + s_ref[1]) + b_ref[...])


def _row_spec(F):
    return pl.BlockSpec((_R, F), lambda i: (i, 0))


def _pair_spec(F):
    return pl.BlockSpec((2, _R, F), lambda i: (0, i, 0))


def _whole_spec(shape):
    return pl.BlockSpec(shape, lambda i: tuple(0 for _ in shape))


def _tc_first(x, w, dgp):
    fin, fout = w.shape
    return pl.pallas_call(
        _tc_first_body,
        grid=(N // _R,),
        in_specs=[_row_spec(fin), _whole_spec((fin, fout)), _pair_spec(16)],
        out_specs=_row_spec(fout),
        out_shape=jax.ShapeDtypeStruct((N, fout), jnp.float32),
    )(x, w, dgp)


def _tc_mid(s, dgp, b, w):
    fin, fout = w.shape
    return pl.pallas_call(
        _tc_mid_body,
        grid=(N // _R,),
        in_specs=[_pair_spec(fin), _pair_spec(16),
                  _whole_spec((1, fin)), _whole_spec((fin, fout))],
        out_specs=_row_spec(fout),
        out_shape=jax.ShapeDtypeStruct((N, fout), jnp.float32),
    )(s, dgp, b, w)


def _tc_last(s, dgp, b):
    fout = s.shape[2]
    return pl.pallas_call(
        _tc_last_body,
        grid=(N // _R,),
        in_specs=[_pair_spec(fout), _pair_spec(16), _whole_spec((1, fout))],
        out_specs=_row_spec(fout),
        out_shape=jax.ShapeDtypeStruct((N, fout), jnp.float32),
    )(s, dgp, b)


@jax.jit
def kernel(features, edge_index, W0, b0, W1, b1, W2, b2):
    edge2 = edge_index.astype(jnp.int32).reshape(2, E // C, C)

    degp = _sc_seg_sum(16, False)(edge2)
    y0 = _tc_first(features, W0, degp)
    s0 = _sc_seg_sum(128, True)(y0, edge2)
    y1 = _tc_mid(s0, degp, b0.reshape(1, -1), W1)
    s1 = _sc_seg_sum(128, True)(y1, edge2)
    y2 = _tc_mid(s1, degp, b1.reshape(1, -1), W2)
    s2 = _sc_seg_sum(64, True)(y2, edge2)
    return _tc_last(s2, degp, b2.reshape(1, -1))


# speedup vs baseline: 28.6321x; 1.0692x over previous
import functools

import jax
import jax.numpy as jnp
from jax import lax
from jax.experimental import pallas as pl
from jax.experimental.pallas import tpu as pltpu
from jax.experimental.pallas import tpu_sc as plsc

N = 10000
NP = 10000
E = 320000
NC = 2
NS = 16
EPT = E // (NC * NS)
ROWS_PER_TILE = NP // NS
C = 125
CH = EPT // C
NB = 4


def _make_sc_seg_sum(F, gather):
    mesh = plsc.VectorSubcoreMesh(core_axis_name="c", subcore_axis_name="s")
    nrows = NB if gather else 1
    dt = jnp.bfloat16 if gather else jnp.float32
    VW = 32 if gather else 16

    scratch = (
        [pltpu.VMEM_SHARED((NP, F), dt)]
        + [pltpu.VMEM((CH, C), jnp.int32)]
        + ([pltpu.VMEM((CH, C), jnp.int32)] if gather else [])
        + [pltpu.VMEM((C, F), dt) for _ in range(nrows)]
        + [pltpu.SemaphoreType.DMA for _ in range(NB * (2 if gather else 1))]
    )

    def body(*refs):
        it = iter(refs)
        if gather:
            y_hbm = next(it)
        edge2 = next(it)
        out_hbm = next(it)
        acc = next(it)
        didx = next(it)
        sidx = next(it) if gather else None
        rows = [next(it) for _ in range(nrows)]
        ssem = [next(it) for _ in range(NB)]
        gsem = [next(it) for _ in range(NB)] if gather else None

        cid = lax.axis_index("c")
        sid = lax.axis_index("s")

        zero = jnp.zeros((VW,), dt)
        one = jnp.ones((VW,), dt)

        def fill(buf, val):
            def fill_row(r, carry):
                for j in range(F // VW):
                    buf[r, pl.ds(j * VW, VW)] = val
                return carry

            lax.fori_loop(0, C, fill_row, 0)

        fill(rows[0], zero)
        rbase = sid * ROWS_PER_TILE
        for t in range(ROWS_PER_TILE // C):
            pltpu.sync_copy(rows[0], acc.at[pl.ds(rbase + t * C, C)])
        remz = ROWS_PER_TILE % C
        if remz:
            pltpu.sync_copy(
                rows[0].at[pl.ds(0, remz)],
                acc.at[pl.ds(rbase + (ROWS_PER_TILE // C) * C, remz)])
        if not gather:
            fill(rows[0], one)
        plsc.subcore_barrier()

        cbase = (cid * NS + sid) * CH
        pltpu.sync_copy(edge2.at[1, pl.ds(cbase, CH)], didx)
        if gather:
            pltpu.sync_copy(edge2.at[0, pl.ds(cbase, CH)], sidx)

        def sstart(i, r, rr):
            pltpu.async_copy(rows[rr], acc.at[didx.at[i]], ssem[r], add=True)

        def swait(r, rr):
            pltpu.make_async_copy(rows[rr], acc.at[didx.at[0]],
                                  ssem[r]).wait()

        if gather:
            def gstart(i, r):
                pltpu.async_copy(y_hbm.at[sidx.at[i]], rows[r], gsem[r])

            def gwait(r):
                pltpu.make_async_copy(y_hbm.at[sidx.at[0]], rows[r],
                                      gsem[r]).wait()

            def step(i, r):
                gwait(r)
                sstart(i, r, r)
                rp = (r + 2) % NB
                swait(rp, rp)
                gstart(i + 2, rp)

            gstart(0, 0)
            gstart(1, 1)
            for i in (0, 1):
                gwait(i)
                sstart(i, i, i)
                gstart(i + 2, i + 2)
            nblocks = (CH - 4) // 4

            def kblock(k, carry):
                b = 2 + 4 * k
                for j in range(4):
                    step(b + j, (2 + j) % NB)
                return carry

            lax.fori_loop(0, nblocks, kblock, 0)
            for i in range(2 + 4 * nblocks, CH - 2):
                step(i, i % NB)
            for i in range(CH - 2, CH):
                gwait(i % NB)
                sstart(i, i % NB, i % NB)
            for i in range(CH - 4, CH):
                swait(i % NB, i % NB)
        else:
            sstart(0, 0, 0)
            sstart(1, 1, 0)
            npairs = (CH - 2) // 2

            def kblock(k, carry):
                i = 2 + 2 * k
                swait(0, 0)
                sstart(i, 0, 0)
                swait(1, 0)
                sstart(i + 1, 1, 0)
                return carry

            lax.fori_loop(0, npairs, kblock, 0)
            for i in range(2 + 2 * npairs, CH):
                p = i & 1
                swait(p, 0)
                sstart(i, p, 0)
            swait(1, 0)
            swait(0, 0)

        plsc.subcore_barrier()

        pltpu.sync_copy(acc.at[pl.ds(rbase, ROWS_PER_TILE)],
                        out_hbm.at[cid, pl.ds(rbase, ROWS_PER_TILE)])

    out_type = jax.ShapeDtypeStruct((NC, NP, F), dt)
    return pl.kernel(body, mesh=mesh, out_type=out_type,
                     scratch_types=scratch,
                     compiler_params=pltpu.CompilerParams(
                         use_tc_tiling_on_sc=False))


_sc_seg_sum = functools.cache(_make_sc_seg_sum)

_R = 2000


def _dinv(dgp):
    deg = dgp[0, :, :1] + dgp[1, :, :1]
    return jnp.where(deg > 0, lax.rsqrt(deg), 0.0)


def _tc_first_body(x_ref, w_ref, dgp_ref, y_ref):
    d = _dinv(dgp_ref[...])
    y = d * jnp.dot(x_ref[...], w_ref[...],
                    preferred_element_type=jnp.float32)
    y_ref[...] = y.astype(jnp.bfloat16)


def _psum(s_ref):
    return (s_ref[0].astype(jnp.float32) + s_ref[1].astype(jnp.float32))


def _tc_mid_body(s_ref, dgp_ref, b_ref, w_ref, y_ref):
    d = _dinv(dgp_ref[...])
    h = jax.nn.relu(d * _psum(s_ref) + b_ref[...])
    y = d * jnp.dot(h, w_ref[...], preferred_element_type=jnp.float32)
    y_ref[...] = y.astype(jnp.bfloat16)


def _tc_last_body(s_ref, dgp_ref, b_ref, y_ref):
    d = _dinv(dgp_ref[...])
    y_ref[...] = jax.nn.relu(d * _psum(s_ref) + b_ref[...])


def _row_spec(F):
    return pl.BlockSpec((_R, F), lambda i: (i, 0))


def _pair_spec(F):
    return pl.BlockSpec((2, _R, F), lambda i: (0, i, 0))


def _whole_spec(shape):
    return pl.BlockSpec(shape, lambda i: tuple(0 for _ in shape))


def _tc_first(x, w, dgp):
    fin, fout = w.shape
    return pl.pallas_call(
        _tc_first_body,
        grid=(N // _R,),
        in_specs=[_row_spec(fin), _whole_spec((fin, fout)), _pair_spec(16)],
        out_specs=_row_spec(fout),
        out_shape=jax.ShapeDtypeStruct((N, fout), jnp.bfloat16),
    )(x, w, dgp)


def _tc_mid(s, dgp, b, w):
    fin, fout = w.shape
    return pl.pallas_call(
        _tc_mid_body,
        grid=(N // _R,),
        in_specs=[_pair_spec(fin), _pair_spec(16),
                  _whole_spec((1, fin)), _whole_spec((fin, fout))],
        out_specs=_row_spec(fout),
        out_shape=jax.ShapeDtypeStruct((N, fout), jnp.bfloat16),
    )(s, dgp, b, w)


def _tc_last(s, dgp, b):
    fout = s.shape[2]
    return pl.pallas_call(
        _tc_last_body,
        grid=(N // _R,),
        in_specs=[_pair_spec(fout), _pair_spec(16), _whole_spec((1, fout))],
        out_specs=_row_spec(fout),
        out_shape=jax.ShapeDtypeStruct((N, fout), jnp.float32),
    )(s, dgp, b)


@jax.jit
def kernel(features, edge_index, W0, b0, W1, b1, W2, b2):
    edge2 = edge_index.astype(jnp.int32).reshape(2, E // C, C)

    degp = _sc_seg_sum(16, False)(edge2)
    y0 = _tc_first(features, W0, degp)
    s0 = _sc_seg_sum(128, True)(y0, edge2)
    y1 = _tc_mid(s0, degp, b0.reshape(1, -1), W1)
    s1 = _sc_seg_sum(128, True)(y1, edge2)
    y2 = _tc_mid(s1, degp, b1.reshape(1, -1), W2)
    s2 = _sc_seg_sum(64, True)(y2, edge2)
    return _tc_last(s2, degp, b2.reshape(1, -1))


# speedup vs baseline: 32.0817x; 1.1205x over previous
import functools

import jax
import jax.numpy as jnp
from jax import lax
from jax.experimental import pallas as pl
from jax.experimental.pallas import tpu as pltpu
from jax.experimental.pallas import tpu_sc as plsc

N = 10000
NP = 10000
E = 320000
NC = 2
NS = 16
EPT = E // (NC * NS)
ROWS_PER_TILE = NP // NS
C = 125
CH = EPT // C
NB = 6


def _make_sc_seg_sum(F, gather):
    mesh = plsc.VectorSubcoreMesh(core_axis_name="c", subcore_axis_name="s")
    nrows = NB if gather else 1
    dt = jnp.bfloat16 if gather else jnp.float32
    VW = 32 if gather else 16

    scratch = (
        [pltpu.VMEM_SHARED((NP, F), dt)]
        + [pltpu.VMEM((CH, C), jnp.int32)]
        + ([pltpu.VMEM((CH, C), jnp.int32)] if gather else [])
        + [pltpu.VMEM((C, F), dt) for _ in range(nrows)]
        + [pltpu.SemaphoreType.DMA for _ in range(NB * (2 if gather else 1))]
    )

    def body(*refs):
        it = iter(refs)
        if gather:
            y_hbm = next(it)
        edge2 = next(it)
        out_hbm = next(it)
        acc = next(it)
        didx = next(it)
        sidx = next(it) if gather else None
        rows = [next(it) for _ in range(nrows)]
        ssem = [next(it) for _ in range(NB)]
        gsem = [next(it) for _ in range(NB)] if gather else None

        cid = lax.axis_index("c")
        sid = lax.axis_index("s")

        zero = jnp.zeros((VW,), dt)
        one = jnp.ones((VW,), dt)

        def fill(buf, val):
            def fill_row(r, carry):
                for j in range(F // VW):
                    buf[r, pl.ds(j * VW, VW)] = val
                return carry

            lax.fori_loop(0, C, fill_row, 0)

        fill(rows[0], zero)
        rbase = sid * ROWS_PER_TILE
        for t in range(ROWS_PER_TILE // C):
            pltpu.sync_copy(rows[0], acc.at[pl.ds(rbase + t * C, C)])
        remz = ROWS_PER_TILE % C
        if remz:
            pltpu.sync_copy(
                rows[0].at[pl.ds(0, remz)],
                acc.at[pl.ds(rbase + (ROWS_PER_TILE // C) * C, remz)])
        if not gather:
            fill(rows[0], one)
        plsc.subcore_barrier()

        cbase = (cid * NS + sid) * CH
        pltpu.sync_copy(edge2.at[1, pl.ds(cbase, CH)], didx)
        if gather:
            pltpu.sync_copy(edge2.at[0, pl.ds(cbase, CH)], sidx)

        def sstart(i, r, rr):
            pltpu.async_copy(rows[rr], acc.at[didx.at[i]], ssem[r], add=True)

        def swait(r, rr):
            pltpu.make_async_copy(rows[rr], acc.at[didx.at[0]],
                                  ssem[r]).wait()

        if gather:
            def gstart(i, r):
                pltpu.async_copy(y_hbm.at[sidx.at[i]], rows[r], gsem[r])

            def gwait(r):
                pltpu.make_async_copy(y_hbm.at[sidx.at[0]], rows[r],
                                      gsem[r]).wait()

            GLAG = NB - 2

            def step(i, r):
                gwait(r)
                sstart(i, r, r)
                rp = (r + GLAG) % NB
                swait(rp, rp)
                gstart(i + GLAG, rp)

            for i in range(GLAG):
                gstart(i, i)
            for i in range(NB - GLAG):
                gwait(i)
                sstart(i, i, i)
                gstart(i + GLAG, i + GLAG)
            first_u = NB - GLAG
            nblocks = (CH - NB) // NB

            def kblock(k, carry):
                b = first_u + NB * k
                for j in range(NB):
                    step(b + j, (first_u + j) % NB)
                return carry

            lax.fori_loop(0, nblocks, kblock, 0)
            for i in range(first_u + NB * nblocks, CH - GLAG):
                step(i, i % NB)
            for i in range(CH - GLAG, CH):
                gwait(i % NB)
                sstart(i, i % NB, i % NB)
            for i in range(CH - NB, CH):
                swait(i % NB, i % NB)
        else:
            sstart(0, 0, 0)
            sstart(1, 1, 0)
            npairs = (CH - 2) // 2

            def kblock(k, carry):
                i = 2 + 2 * k
                swait(0, 0)
                sstart(i, 0, 0)
                swait(1, 0)
                sstart(i + 1, 1, 0)
                return carry

            lax.fori_loop(0, npairs, kblock, 0)
            for i in range(2 + 2 * npairs, CH):
                p = i & 1
                swait(p, 0)
                sstart(i, p, 0)
            swait(1, 0)
            swait(0, 0)

        plsc.subcore_barrier()

        pltpu.sync_copy(acc.at[pl.ds(rbase, ROWS_PER_TILE)],
                        out_hbm.at[cid, pl.ds(rbase, ROWS_PER_TILE)])

    out_type = jax.ShapeDtypeStruct((NC, NP, F), dt)
    return pl.kernel(body, mesh=mesh, out_type=out_type,
                     scratch_types=scratch,
                     compiler_params=pltpu.CompilerParams(
                         use_tc_tiling_on_sc=False))


_sc_seg_sum = functools.cache(_make_sc_seg_sum)

_R = 2000


def _dinv(dgp):
    deg = dgp[0, :, :1] + dgp[1, :, :1]
    return jnp.where(deg > 0, lax.rsqrt(deg), 0.0)


def _tc_first_body(x_ref, w_ref, dgp_ref, y_ref):
    d = _dinv(dgp_ref[...])
    y = d * jnp.dot(x_ref[...], w_ref[...],
                    preferred_element_type=jnp.float32)
    y_ref[...] = y.astype(jnp.bfloat16)


def _psum(s_ref):
    return (s_ref[0].astype(jnp.float32) + s_ref[1].astype(jnp.float32))


def _tc_mid_body(s_ref, dgp_ref, b_ref, w_ref, y_ref):
    d = _dinv(dgp_ref[...])
    h = jax.nn.relu(d * _psum(s_ref) + b_ref[...])
    y = d * jnp.dot(h, w_ref[...], preferred_element_type=jnp.float32)
    y_ref[...] = y.astype(jnp.bfloat16)


def _tc_last_body(s_ref, dgp_ref, b_ref, y_ref):
    d = _dinv(dgp_ref[...])
    y_ref[...] = jax.nn.relu(d * _psum(s_ref) + b_ref[...])


def _row_spec(F):
    return pl.BlockSpec((_R, F), lambda i: (i, 0))


def _pair_spec(F):
    return pl.BlockSpec((2, _R, F), lambda i: (0, i, 0))


def _whole_spec(shape):
    return pl.BlockSpec(shape, lambda i: tuple(0 for _ in shape))


def _tc_first(x, w, dgp):
    fin, fout = w.shape
    return pl.pallas_call(
        _tc_first_body,
        grid=(N // _R,),
        in_specs=[_row_spec(fin), _whole_spec((fin, fout)), _pair_spec(16)],
        out_specs=_row_spec(fout),
        out_shape=jax.ShapeDtypeStruct((N, fout), jnp.bfloat16),
    )(x, w, dgp)


def _tc_mid(s, dgp, b, w):
    fin, fout = w.shape
    return pl.pallas_call(
        _tc_mid_body,
        grid=(N // _R,),
        in_specs=[_pair_spec(fin), _pair_spec(16),
                  _whole_spec((1, fin)), _whole_spec((fin, fout))],
        out_specs=_row_spec(fout),
        out_shape=jax.ShapeDtypeStruct((N, fout), jnp.bfloat16),
    )(s, dgp, b, w)


def _tc_last(s, dgp, b):
    fout = s.shape[2]
    return pl.pallas_call(
        _tc_last_body,
        grid=(N // _R,),
        in_specs=[_pair_spec(fout), _pair_spec(16), _whole_spec((1, fout))],
        out_specs=_row_spec(fout),
        out_shape=jax.ShapeDtypeStruct((N, fout), jnp.float32),
    )(s, dgp, b)


@jax.jit
def kernel(features, edge_index, W0, b0, W1, b1, W2, b2):
    edge2 = edge_index.astype(jnp.int32).reshape(2, E // C, C)

    degp = _sc_seg_sum(16, False)(edge2)
    y0 = _tc_first(features, W0, degp)
    s0 = _sc_seg_sum(128, True)(y0, edge2)
    y1 = _tc_mid(s0, degp, b0.reshape(1, -1), W1)
    s1 = _sc_seg_sum(128, True)(y1, edge2)
    y2 = _tc_mid(s1, degp, b1.reshape(1, -1), W2)
    s2 = _sc_seg_sum(64, True)(y2, edge2)
    return _tc_last(s2, degp, b2.reshape(1, -1))


# speedup vs baseline: 32.2532x; 1.0053x over previous
import functools

import jax
import jax.numpy as jnp
from jax import lax
from jax.experimental import pallas as pl
from jax.experimental.pallas import tpu as pltpu
from jax.experimental.pallas import tpu_sc as plsc

N = 10000
NP = 10000
E = 320000
NC = 2
NS = 16
EPT = E // (NC * NS)
ROWS_PER_TILE = NP // NS
C = 125
CH = EPT // C
NB = 8


def _make_sc_seg_sum(F, gather):
    mesh = plsc.VectorSubcoreMesh(core_axis_name="c", subcore_axis_name="s")
    nrows = NB if gather else 1
    dt = jnp.bfloat16 if gather else jnp.float32
    VW = 32 if gather else 16

    scratch = (
        [pltpu.VMEM_SHARED((NP, F), dt)]
        + [pltpu.VMEM((CH, C), jnp.int32)]
        + ([pltpu.VMEM((CH, C), jnp.int32)] if gather else [])
        + [pltpu.VMEM((C, F), dt) for _ in range(nrows)]
        + [pltpu.SemaphoreType.DMA for _ in range(NB * (2 if gather else 1))]
    )

    def body(*refs):
        it = iter(refs)
        if gather:
            y_hbm = next(it)
        edge2 = next(it)
        out_hbm = next(it)
        acc = next(it)
        didx = next(it)
        sidx = next(it) if gather else None
        rows = [next(it) for _ in range(nrows)]
        ssem = [next(it) for _ in range(NB)]
        gsem = [next(it) for _ in range(NB)] if gather else None

        cid = lax.axis_index("c")
        sid = lax.axis_index("s")

        zero = jnp.zeros((VW,), dt)
        one = jnp.ones((VW,), dt)

        def fill(buf, val):
            def fill_row(r, carry):
                for j in range(F // VW):
                    buf[r, pl.ds(j * VW, VW)] = val
                return carry

            lax.fori_loop(0, C, fill_row, 0)

        fill(rows[0], zero)
        rbase = sid * ROWS_PER_TILE
        for t in range(ROWS_PER_TILE // C):
            pltpu.sync_copy(rows[0], acc.at[pl.ds(rbase + t * C, C)])
        remz = ROWS_PER_TILE % C
        if remz:
            pltpu.sync_copy(
                rows[0].at[pl.ds(0, remz)],
                acc.at[pl.ds(rbase + (ROWS_PER_TILE // C) * C, remz)])
        if not gather:
            fill(rows[0], one)
        plsc.subcore_barrier()

        cbase = (cid * NS + sid) * CH
        pltpu.sync_copy(edge2.at[1, pl.ds(cbase, CH)], didx)
        if gather:
            pltpu.sync_copy(edge2.at[0, pl.ds(cbase, CH)], sidx)

        def sstart(i, r, rr):
            pltpu.async_copy(rows[rr], acc.at[didx.at[i]], ssem[r], add=True)

        def swait(r, rr):
            pltpu.make_async_copy(rows[rr], acc.at[didx.at[0]],
                                  ssem[r]).wait()

        if gather:
            def gstart(i, r):
                pltpu.async_copy(y_hbm.at[sidx.at[i]], rows[r], gsem[r])

            def gwait(r):
                pltpu.make_async_copy(y_hbm.at[sidx.at[0]], rows[r],
                                      gsem[r]).wait()

            GLAG = NB - 2

            def step(i, r):
                gwait(r)
                sstart(i, r, r)
                rp = (r + GLAG) % NB
                swait(rp, rp)
                gstart(i + GLAG, rp)

            for i in range(GLAG):
                gstart(i, i)
            for i in range(NB - GLAG):
                gwait(i)
                sstart(i, i, i)
                gstart(i + GLAG, i + GLAG)
            first_u = NB - GLAG
            nblocks = (CH - NB) // NB

            def kblock(k, carry):
                b = first_u + NB * k
                for j in range(NB):
                    step(b + j, (first_u + j) % NB)
                return carry

            lax.fori_loop(0, nblocks, kblock, 0)
            for i in range(first_u + NB * nblocks, CH - GLAG):
                step(i, i % NB)
            for i in range(CH - GLAG, CH):
                gwait(i % NB)
                sstart(i, i % NB, i % NB)
            for i in range(CH - NB, CH):
                swait(i % NB, i % NB)
        else:
            sstart(0, 0, 0)
            sstart(1, 1, 0)
            npairs = (CH - 2) // 2

            def kblock(k, carry):
                i = 2 + 2 * k
                swait(0, 0)
                sstart(i, 0, 0)
                swait(1, 0)
                sstart(i + 1, 1, 0)
                return carry

            lax.fori_loop(0, npairs, kblock, 0)
            for i in range(2 + 2 * npairs, CH):
                p = i & 1
                swait(p, 0)
                sstart(i, p, 0)
            swait(1, 0)
            swait(0, 0)

        plsc.subcore_barrier()

        pltpu.sync_copy(acc.at[pl.ds(rbase, ROWS_PER_TILE)],
                        out_hbm.at[cid, pl.ds(rbase, ROWS_PER_TILE)])

    out_type = jax.ShapeDtypeStruct((NC, NP, F), dt)
    return pl.kernel(body, mesh=mesh, out_type=out_type,
                     scratch_types=scratch,
                     compiler_params=pltpu.CompilerParams(
                         use_tc_tiling_on_sc=False))


_sc_seg_sum = functools.cache(_make_sc_seg_sum)

_R = 2000


def _dinv(dgp):
    deg = dgp[0, :, :1] + dgp[1, :, :1]
    return jnp.where(deg > 0, lax.rsqrt(deg), 0.0)


def _tc_first_body(x_ref, w_ref, dgp_ref, y_ref):
    d = _dinv(dgp_ref[...])
    y = d * jnp.dot(x_ref[...], w_ref[...],
                    preferred_element_type=jnp.float32)
    y_ref[...] = y.astype(jnp.bfloat16)


def _psum(s_ref):
    return (s_ref[0].astype(jnp.float32) + s_ref[1].astype(jnp.float32))


def _tc_mid_body(s_ref, dgp_ref, b_ref, w_ref, y_ref):
    d = _dinv(dgp_ref[...])
    h = jax.nn.relu(d * _psum(s_ref) + b_ref[...])
    y = d * jnp.dot(h, w_ref[...], preferred_element_type=jnp.float32)
    y_ref[...] = y.astype(jnp.bfloat16)


def _tc_last_body(s_ref, dgp_ref, b_ref, y_ref):
    d = _dinv(dgp_ref[...])
    y_ref[...] = jax.nn.relu(d * _psum(s_ref) + b_ref[...])


def _row_spec(F):
    return pl.BlockSpec((_R, F), lambda i: (i, 0))


def _pair_spec(F):
    return pl.BlockSpec((2, _R, F), lambda i: (0, i, 0))


def _whole_spec(shape):
    return pl.BlockSpec(shape, lambda i: tuple(0 for _ in shape))


def _tc_first(x, w, dgp):
    fin, fout = w.shape
    return pl.pallas_call(
        _tc_first_body,
        grid=(N // _R,),
        in_specs=[_row_spec(fin), _whole_spec((fin, fout)), _pair_spec(16)],
        out_specs=_row_spec(fout),
        out_shape=jax.ShapeDtypeStruct((N, fout), jnp.bfloat16),
    )(x, w, dgp)


def _tc_mid(s, dgp, b, w):
    fin, fout = w.shape
    return pl.pallas_call(
        _tc_mid_body,
        grid=(N // _R,),
        in_specs=[_pair_spec(fin), _pair_spec(16),
                  _whole_spec((1, fin)), _whole_spec((fin, fout))],
        out_specs=_row_spec(fout),
        out_shape=jax.ShapeDtypeStruct((N, fout), jnp.bfloat16),
    )(s, dgp, b, w)


def _tc_last(s, dgp, b):
    fout = s.shape[2]
    return pl.pallas_call(
        _tc_last_body,
        grid=(N // _R,),
        in_specs=[_pair_spec(fout), _pair_spec(16), _whole_spec((1, fout))],
        out_specs=_row_spec(fout),
        out_shape=jax.ShapeDtypeStruct((N, fout), jnp.float32),
    )(s, dgp, b)


@jax.jit
def kernel(features, edge_index, W0, b0, W1, b1, W2, b2):
    edge2 = edge_index.astype(jnp.int32).reshape(2, E // C, C)

    degp = _sc_seg_sum(16, False)(edge2)
    y0 = _tc_first(features, W0, degp)
    s0 = _sc_seg_sum(128, True)(y0, edge2)
    y1 = _tc_mid(s0, degp, b0.reshape(1, -1), W1)
    s1 = _sc_seg_sum(128, True)(y1, edge2)
    y2 = _tc_mid(s1, degp, b1.reshape(1, -1), W2)
    s2 = _sc_seg_sum(64, True)(y2, edge2)
    return _tc_last(s2, degp, b2.reshape(1, -1))
